# baseline (speedup 1.0000x reference)
import functools

import jax
import jax.numpy as jnp
from jax import lax
from jax.experimental import pallas as pl
from jax.experimental.pallas import tpu as pltpu
from jax.experimental.pallas import tpu_sc as plsc

D_V = 1_000_000
D_M = 64
B = 16384

NC = 2
NS = 16
NW = NC * NS
SEG = B // NW


def _body(idx_hbm, tableT_hbm, out_hbm, idx_v, rows_v, sem, gsem):
    cid = lax.axis_index("c")
    sid = lax.axis_index("s")
    wid = sid * NC + cid
    base = wid * SEG

    pltpu.sync_copy(idx_hbm.at[pl.ds(base, SEG)], idx_v)

    src = tableT_hbm.at[idx_v]
    pltpu.make_async_copy(src, rows_v, gsem).start()
    pltpu.make_async_copy(src, rows_v, gsem).wait()

    pltpu.sync_copy(rows_v, out_hbm.at[pl.ds(base, SEG), :])


def kernel(token_indices, lookup):
    idx = token_indices.astype(jnp.int32)
    tableT = jnp.transpose(lookup)
    mesh = plsc.VectorSubcoreMesh(core_axis_name="c", subcore_axis_name="s")
    k = functools.partial(
        pl.kernel,
        mesh=mesh,
        out_type=jax.ShapeDtypeStruct((B, D_M), jnp.float32),
        scratch_types=[
            pltpu.VMEM((SEG,), jnp.int32),
            pltpu.VMEM((SEG, D_M), jnp.float32),
            pltpu.SemaphoreType.DMA,
            pltpu.SemaphoreType.DMA,
        ],
        compiler_params=pltpu.CompilerParams(use_tc_tiling_on_sc=False),
    )(_body)
    rows = k(idx, tableT)
    return rows.T


# speedup vs baseline: 1.1141x; 1.1141x over previous
import functools

import jax
import jax.numpy as jnp
from jax import lax
from jax.experimental import pallas as pl
from jax.experimental.pallas import tpu as pltpu
from jax.experimental.pallas import tpu_sc as plsc

D_V = 1_000_000
D_M = 64
B = 16384

NC = 2
NS = 16
NW = NC * NS
SEG = B // NW


def _body(idx_hbm, tableT_hbm, out_hbm, idx_v, rows_v, sem, gsem):
    cid = lax.axis_index("c")
    sid = lax.axis_index("s")
    wid = sid * NC + cid
    base = wid * SEG

    pltpu.sync_copy(idx_hbm.at[pl.ds(base, SEG)], idx_v)

    src = tableT_hbm.at[idx_v]
    pltpu.make_async_copy(src, rows_v, gsem).start()
    pltpu.make_async_copy(src, rows_v, gsem).wait()

    pltpu.sync_copy(rows_v, out_hbm.at[pl.ds(base, SEG), :])


def kernel(token_indices, lookup):
    idx = token_indices.astype(jnp.int32)
    tableT = jnp.pad(jnp.transpose(lookup), ((0, 0), (0, D_M)))
    mesh = plsc.VectorSubcoreMesh(core_axis_name="c", subcore_axis_name="s")
    k = functools.partial(
        pl.kernel,
        mesh=mesh,
        out_type=jax.ShapeDtypeStruct((B, 2 * D_M), jnp.float32),
        scratch_types=[
            pltpu.VMEM((SEG,), jnp.int32),
            pltpu.VMEM((SEG, 2 * D_M), jnp.float32),
            pltpu.SemaphoreType.DMA,
            pltpu.SemaphoreType.DMA,
        ],
    )(_body)
    rows = k(idx, tableT)
    return rows[:, :D_M].T
